# baseline (speedup 1.0000x reference)
import functools

import jax
import jax.numpy as jnp
from jax import lax
from jax.experimental import pallas as pl
from jax.experimental.pallas import tpu as pltpu
from jax.experimental.pallas import tpu_sc as plsc

N_NODES = 10000
D_FEAT = 128
HIDDEN = 64
NCLS = 16
K_HOPS = 10
ALPHA = 0.1
OMA = 1.0 - ALPHA

NPAD = 10240
NTILES = 16
ROWS_PT = NPAD // NTILES
HALF = ROWS_PT // 2
E = 320000
CHUNK = 128
CH_PT = 160
E_PT = CH_PT * CHUNK
E_PAD = NTILES * E_PT
V16 = 16
KB = 5
NDROWS = NPAD // V16
DROWS_PT = NDROWS // NTILES
DCHUNK = 128
DCH = NDROWS // DCHUNK


def _mlp_body(x_ref, w1_ref, b1_ref, w2_ref, b2_ref, h_ref):
    a = jnp.dot(x_ref[...], w1_ref[...], preferred_element_type=jnp.float32)
    a = jnp.maximum(a + b1_ref[...][None, :], 0.0)
    h_ref[...] = (
        jnp.dot(a, w2_ref[...], preferred_element_type=jnp.float32)
        + b2_ref[...][None, :]
    )


def _mlp(xp, W1, b1, W2, b2):
    return pl.pallas_call(
        _mlp_body,
        out_shape=jax.ShapeDtypeStruct((NPAD, NCLS), jnp.float32),
    )(xp, W1, b1, W2, b2)


_SC_MESH = plsc.VectorSubcoreMesh(core_axis_name="c", subcore_axis_name="s")


@functools.partial(
    pl.kernel,
    out_type=[
        jax.ShapeDtypeStruct((NPAD, NCLS), jnp.float32),
        jax.ShapeDtypeStruct((NPAD,), jnp.float32),
    ],
    mesh=_SC_MESH,
    compiler_params=pltpu.CompilerParams(
        needs_layout_passes=False, use_tc_tiling_on_sc=False),
    scratch_types=[
        pltpu.HBM((2 * NPAD, NCLS), jnp.float32),
        pltpu.VMEM_SHARED((NPAD, NCLS), jnp.float32),
        pltpu.VMEM_SHARED((NDROWS, V16), jnp.float32),
        pltpu.VMEM((CH_PT, CHUNK), jnp.int32),
        pltpu.VMEM((CH_PT, CHUNK), jnp.int32),
        pltpu.VMEM((ROWS_PT, NCLS), jnp.float32),
        pltpu.VMEM((ROWS_PT, NCLS), jnp.float32),
        pltpu.VMEM((ROWS_PT, NCLS), jnp.float32),
        pltpu.VMEM((ROWS_PT, NCLS), jnp.float32),
        pltpu.VMEM((2 * KB, CHUNK, NCLS), jnp.float32),
        pltpu.VMEM((DCH, DCHUNK), jnp.int32),
        pltpu.VMEM((DROWS_PT, V16), jnp.float32),
        pltpu.VMEM((ROWS_PT,), jnp.float32),
        pltpu.VMEM((ROWS_PT,), jnp.float32),
        pltpu.SemaphoreType.DMA,
        pltpu.SemaphoreType.DMA((2,)),
    ],
)
def _prop(h_hbm, src_hbm, dst_hbm, zk_hbm, deg_hbm,
          z_hb, agg_sh, deg_sh,
          src_t, dst_t, hbuf, cbuf, gbuf, zbuf, ebuf, identx, dtmp, tmp,
          dinvb, gsem, ssem):
    abuf = hbuf
    cid = lax.axis_index("c")
    tid = lax.axis_index("s")
    row0 = tid * ROWS_PT

    ones16 = jnp.ones((V16,), jnp.float32)
    zeros16 = jnp.zeros((V16,), jnp.float32)
    iota16 = lax.iota(jnp.int32, V16)

    pltpu.sync_copy(src_hbm.at[tid], src_t)
    pltpu.sync_copy(dst_hbm.at[tid], dst_t)
    pltpu.sync_copy(h_hbm.at[pl.ds(row0, ROWS_PT)], hbuf)

    cidN = cid * NPAD

    def _src_off(i, _):
        for j in range(CHUNK // V16):
            sl = pl.ds(j * V16, V16)
            src_t[i, sl] = src_t[i, sl] + cidN
        return 0
    lax.fori_loop(0, CH_PT, _src_off, 0)

    for c in range(DCH):
        for j in range(DCHUNK // V16):
            identx[c, pl.ds(j * V16, V16)] = iota16 + (c * DCHUNK + j * V16)

    def _zero_hist(r, _):
        cbuf[r, :] = zeros16
        return 0
    lax.fori_loop(0, ROWS_PT, _zero_hist, 0)

    def _zero_own(r, _):
        dtmp[r, :] = zeros16
        return 0
    lax.fori_loop(0, DROWS_PT, _zero_own, 0)
    pltpu.sync_copy(dtmp, deg_sh.at[pl.ds(tid * DROWS_PT, DROWS_PT)])

    def _count(i, _):
        for j in range(CHUNK // V16):
            idx = dst_t[i, pl.ds(j * V16, V16)]
            plsc.addupdate_scatter(cbuf, [idx >> 4, idx & 15], ones16)
        return 0
    lax.fori_loop(0, CH_PT, _count, 0)

    plsc.subcore_barrier()
    for c in range(DCH):
        pltpu.sync_copy(cbuf.at[pl.ds(c * DCHUNK, DCHUNK)],
                        deg_sh.at[identx.at[c]], add=True)
    plsc.subcore_barrier()

    pltpu.sync_copy(deg_sh.at[pl.ds(tid * DROWS_PT, DROWS_PT)], dtmp)

    def _deg_own(r, _):
        tmp[pl.ds(r * V16, V16)] = dtmp[r, :] + 1.0
        return 0
    lax.fori_loop(0, DROWS_PT, _deg_own, 0)

    off = cid * HALF
    pltpu.sync_copy(tmp.at[pl.ds(off, HALF)], deg_hbm.at[pl.ds(row0 + off, HALF)])

    def _rsqrt(i, _):
        s = pl.ds(i * V16, V16)
        d = tmp[s]
        yi = jnp.int32(0x5F3759DF) - (plsc.bitcast(d, jnp.int32) >> 1)
        y = plsc.bitcast(yi, jnp.float32)
        hx = d * 0.5
        y = y * (1.5 - hx * y * y)
        y = y * (1.5 - hx * y * y)
        y = y * (1.5 - hx * y * y)
        dinvb[s] = y
        return 0
    lax.fori_loop(0, ROWS_PT // V16, _rsqrt, 0)

    def _init_row(r, _):
        iv = jnp.full((V16,), r, jnp.int32)
        dv = plsc.load_gather(dinvb, [iv])
        hrow = hbuf[r, :]
        z0 = dv * hrow
        zbuf[r, :] = z0
        cbuf[r, :] = (dv * dv) * OMA
        gbuf[r, :] = z0 * ALPHA
        return 0
    lax.fori_loop(0, ROWS_PT, _init_row, 0)

    pltpu.sync_copy(zbuf, z_hb.at[pl.ds(cidN + row0, ROWS_PT)])
    pltpu.sync_copy(zbuf, agg_sh.at[pl.ds(row0, ROWS_PT)])

    for k in range(K_HOPS):
        plsc.subcore_barrier()

        def _pair(gg, _):
            for p in range(2):
                g = gg * 2 + p

                @pl.when(gg > 0)
                def _drain_prev():
                    for b in range(KB):
                        pltpu.make_async_copy(
                            h_hbm.at[pl.ds(0, CHUNK)],
                            ebuf.at[p * KB + b], ssem.at[p]).wait()

                handles = []
                for b in range(KB):
                    c = g * KB + b
                    handles.append(pltpu.async_copy(
                        z_hb.at[src_t.at[c]], ebuf.at[p * KB + b], gsem))
                for h in handles:
                    h.wait()
                for b in range(KB):
                    c = g * KB + b
                    pltpu.async_copy(ebuf.at[p * KB + b],
                                     agg_sh.at[dst_t.at[c]], ssem.at[p],
                                     add=True)
            return 0
        lax.fori_loop(0, CH_PT // (2 * KB), _pair, 0)

        for p in range(2):
            for b in range(KB):
                pltpu.make_async_copy(h_hbm.at[pl.ds(0, CHUNK)],
                                      ebuf.at[p * KB + b], ssem.at[p]).wait()

        plsc.subcore_barrier()

        pltpu.sync_copy(agg_sh.at[pl.ds(row0, ROWS_PT)], abuf)

        def _dense(r, _):
            zbuf[r, :] = cbuf[r, :] * abuf[r, :] + gbuf[r, :]
            return 0
        lax.fori_loop(0, ROWS_PT, _dense, 0)

        if k + 1 < K_HOPS:
            pltpu.sync_copy(zbuf, z_hb.at[pl.ds(cidN + row0, ROWS_PT)])
            pltpu.sync_copy(zbuf, agg_sh.at[pl.ds(row0, ROWS_PT)])

    pltpu.sync_copy(zbuf.at[pl.ds(off, HALF)], zk_hbm.at[pl.ds(row0 + off, HALF)])


def _lsm_body(z_ref, deg_ref, o_ref):
    x = z_ref[...] * jnp.sqrt(deg_ref[...])
    m = jnp.max(x, axis=1, keepdims=True)
    e = jnp.exp(x - m)
    o_ref[...] = x - m - jnp.log(jnp.sum(e, axis=1, keepdims=True))


def _lsm(zk, deg2):
    return pl.pallas_call(
        _lsm_body,
        out_shape=jax.ShapeDtypeStruct((N_NODES, NCLS), jnp.float32),
    )(zk, deg2)


def kernel(x, edge_index, W1, b1, W2, b2):
    xp = jnp.pad(x, ((0, NPAD - N_NODES), (0, 0)))
    h = _mlp(xp, W1, b1, W2, b2)

    src = jnp.asarray(edge_index[0], jnp.int32)
    dst = jnp.asarray(edge_index[1], jnp.int32)
    pad = jnp.full((E_PAD - E,), NPAD - 1, jnp.int32)
    src3 = jnp.concatenate([src, pad]).reshape(NTILES, CH_PT, CHUNK)
    dst3 = jnp.concatenate([dst, pad]).reshape(NTILES, CH_PT, CHUNK)

    zk, deg = _prop(h, src3, dst3)
    return _lsm(zk[:N_NODES], deg[:N_NODES].reshape(N_NODES, 1))


# speedup vs baseline: 1.7621x; 1.7621x over previous
import functools

import jax
import jax.numpy as jnp
from jax import lax
from jax.experimental import pallas as pl
from jax.experimental.pallas import tpu as pltpu
from jax.experimental.pallas import tpu_sc as plsc

N_NODES = 10000
D_FEAT = 128
HIDDEN = 64
NCLS = 16
K_HOPS = 10
ALPHA = 0.1
OMA = 1.0 - ALPHA

NPAD = 10240
NTILES = 16
ROWS_PT = NPAD // NTILES
HALF = ROWS_PT // 2
E = 320000
CHUNK = 256
CH_PT = 80
E_PT = CH_PT * CHUNK
E_PAD = NTILES * E_PT
V16 = 16
KB = 2
NDROWS = NPAD // V16
DROWS_PT = NDROWS // NTILES
DCHUNK = 128
DCH = NDROWS // DCHUNK


MLP_BLK = 400
MLP_NREAL = N_NODES // MLP_BLK
MLP_NBLK = NPAD // MLP_BLK


def _mlp_body(x_ref, w1_ref, b1_ref, w2_ref, b2_ref, h_ref):
    a = jnp.dot(x_ref[...], w1_ref[...], preferred_element_type=jnp.float32)
    a = jnp.maximum(a + b1_ref[...][None, :], 0.0)
    h_ref[...] = (
        jnp.dot(a, w2_ref[...], preferred_element_type=jnp.float32)
        + b2_ref[...][None, :]
    )


def _mlp(x, W1, b1, W2, b2):
    nblk = NPAD // 80
    last = N_NODES // 80 - 1
    return pl.pallas_call(
        _mlp_body,
        grid=(nblk,),
        in_specs=[
            pl.BlockSpec((80, D_FEAT), lambda i: (jnp.minimum(i, last), 0)),
            pl.BlockSpec((D_FEAT, HIDDEN), lambda i: (0, 0)),
            pl.BlockSpec((HIDDEN,), lambda i: (0,)),
            pl.BlockSpec((HIDDEN, NCLS), lambda i: (0, 0)),
            pl.BlockSpec((NCLS,), lambda i: (0,)),
        ],
        out_specs=pl.BlockSpec((80, NCLS), lambda i: (i, 0)),
        out_shape=jax.ShapeDtypeStruct((NPAD, NCLS), jnp.float32),
    )(x, W1, b1, W2, b2)


_SC_MESH = plsc.VectorSubcoreMesh(core_axis_name="c", subcore_axis_name="s")


@functools.partial(
    pl.kernel,
    out_type=[
        jax.ShapeDtypeStruct((NPAD, NCLS), jnp.float32),
        jax.ShapeDtypeStruct((NPAD,), jnp.float32),
    ],
    mesh=_SC_MESH,
    compiler_params=pltpu.CompilerParams(
        needs_layout_passes=False, use_tc_tiling_on_sc=False),
    scratch_types=[
        pltpu.VMEM_SHARED((NPAD, NCLS), jnp.float32),
        pltpu.VMEM_SHARED((NPAD, NCLS), jnp.float32),
        pltpu.VMEM_SHARED((NDROWS, V16), jnp.float32),
        pltpu.VMEM((CH_PT, CHUNK), jnp.int32),
        pltpu.VMEM((CH_PT, CHUNK), jnp.int32),
        pltpu.VMEM((ROWS_PT, NCLS), jnp.float32),
        pltpu.VMEM((ROWS_PT, NCLS), jnp.float32),
        pltpu.VMEM((ROWS_PT, NCLS), jnp.float32),
        pltpu.VMEM((ROWS_PT, NCLS), jnp.float32),
        pltpu.VMEM((2 * KB, CHUNK, NCLS), jnp.float32),
        pltpu.VMEM((DCH, DCHUNK), jnp.int32),
        pltpu.VMEM((DROWS_PT, V16), jnp.float32),
        pltpu.VMEM((ROWS_PT,), jnp.float32),
        pltpu.VMEM((ROWS_PT,), jnp.float32),
        pltpu.SemaphoreType.DMA,
        pltpu.SemaphoreType.DMA((2,)),
    ],
)
def _prop(h_hbm, src_hbm, dst_hbm, zk_hbm, deg_hbm,
          z_sh, agg_sh, deg_sh,
          src_t, dst_t, hbuf, cbuf, gbuf, zbuf, ebuf, identx, dtmp, tmp,
          dinvb, gsem, ssem):
    abuf = hbuf
    cid = lax.axis_index("c")
    tid = lax.axis_index("s")
    row0 = tid * ROWS_PT

    ones16 = jnp.ones((V16,), jnp.float32)
    zeros16 = jnp.zeros((V16,), jnp.float32)
    iota16 = lax.iota(jnp.int32, V16)

    pltpu.sync_copy(src_hbm.at[tid], src_t)
    pltpu.sync_copy(dst_hbm.at[tid], dst_t)
    pltpu.sync_copy(h_hbm.at[pl.ds(row0, ROWS_PT)], hbuf)

    for c in range(DCH):
        for j in range(DCHUNK // V16):
            identx[c, pl.ds(j * V16, V16)] = iota16 + (c * DCHUNK + j * V16)

    def _zero_hist(r, _):
        cbuf[r, :] = zeros16
        return 0
    lax.fori_loop(0, ROWS_PT, _zero_hist, 0)

    def _zero_own(r, _):
        dtmp[r, :] = zeros16
        return 0
    lax.fori_loop(0, DROWS_PT, _zero_own, 0)
    pltpu.sync_copy(dtmp, deg_sh.at[pl.ds(tid * DROWS_PT, DROWS_PT)])

    def _count(i, _):
        for j in range(CHUNK // V16):
            idx = dst_t[i, pl.ds(j * V16, V16)]
            plsc.addupdate_scatter(cbuf, [idx >> 4, idx & 15], ones16)
        return 0
    lax.fori_loop(0, CH_PT, _count, 0)

    plsc.subcore_barrier()
    for c in range(DCH):
        pltpu.sync_copy(cbuf.at[pl.ds(c * DCHUNK, DCHUNK)],
                        deg_sh.at[identx.at[c]], add=True)
    plsc.subcore_barrier()

    pltpu.sync_copy(deg_sh.at[pl.ds(tid * DROWS_PT, DROWS_PT)], dtmp)

    def _deg_own(r, _):
        tmp[pl.ds(r * V16, V16)] = dtmp[r, :] + 1.0
        return 0
    lax.fori_loop(0, DROWS_PT, _deg_own, 0)

    off = cid * HALF
    pltpu.sync_copy(tmp.at[pl.ds(off, HALF)], deg_hbm.at[pl.ds(row0 + off, HALF)])

    def _rsqrt(i, _):
        s = pl.ds(i * V16, V16)
        d = tmp[s]
        yi = jnp.int32(0x5F3759DF) - (plsc.bitcast(d, jnp.int32) >> 1)
        y = plsc.bitcast(yi, jnp.float32)
        hx = d * 0.5
        y = y * (1.5 - hx * y * y)
        y = y * (1.5 - hx * y * y)
        y = y * (1.5 - hx * y * y)
        dinvb[s] = y
        return 0
    lax.fori_loop(0, ROWS_PT // V16, _rsqrt, 0)

    def _init_row(r, _):
        iv = jnp.full((V16,), r, jnp.int32)
        dv = plsc.load_gather(dinvb, [iv])
        hrow = hbuf[r, :]
        z0 = dv * hrow
        zbuf[r, :] = z0
        cbuf[r, :] = (dv * dv) * OMA
        gbuf[r, :] = z0 * ALPHA
        return 0
    lax.fori_loop(0, ROWS_PT, _init_row, 0)

    pltpu.sync_copy(zbuf, z_sh.at[pl.ds(row0, ROWS_PT)])
    pltpu.sync_copy(zbuf, agg_sh.at[pl.ds(row0, ROWS_PT)])

    for k in range(K_HOPS):
        plsc.subcore_barrier()

        def _pair(gg, _):
            for p in range(2):
                g = gg * 2 + p

                @pl.when(gg > 0)
                def _drain_prev():
                    for b in range(KB):
                        pltpu.make_async_copy(
                            h_hbm.at[pl.ds(0, CHUNK)],
                            ebuf.at[p * KB + b], ssem.at[p]).wait()

                handles = []
                for b in range(KB):
                    c = g * KB + b
                    handles.append(pltpu.async_copy(
                        z_sh.at[src_t.at[c]], ebuf.at[p * KB + b], gsem))
                for h in handles:
                    h.wait()
                for b in range(KB):
                    c = g * KB + b
                    pltpu.async_copy(ebuf.at[p * KB + b],
                                     agg_sh.at[dst_t.at[c]], ssem.at[p],
                                     add=True)
            return 0
        lax.fori_loop(0, CH_PT // (2 * KB), _pair, 0)

        for p in range(2):
            for b in range(KB):
                pltpu.make_async_copy(h_hbm.at[pl.ds(0, CHUNK)],
                                      ebuf.at[p * KB + b], ssem.at[p]).wait()

        plsc.subcore_barrier()

        pltpu.sync_copy(agg_sh.at[pl.ds(row0, ROWS_PT)], abuf)

        def _dense(r4, _):
            for u in range(4):
                r = r4 * 4 + u
                zbuf[r, :] = cbuf[r, :] * abuf[r, :] + gbuf[r, :]
            return 0
        lax.fori_loop(0, ROWS_PT // 4, _dense, 0)

        if k + 1 < K_HOPS:
            w1 = pltpu.async_copy(zbuf, z_sh.at[pl.ds(row0, ROWS_PT)], gsem)
            w2 = pltpu.async_copy(zbuf, agg_sh.at[pl.ds(row0, ROWS_PT)], gsem)
            w1.wait()
            w2.wait()

    pltpu.sync_copy(zbuf.at[pl.ds(off, HALF)], zk_hbm.at[pl.ds(row0 + off, HALF)])


def _lsm_body(z_ref, deg_ref, o_ref):
    x = z_ref[...] * jnp.sqrt(deg_ref[...])
    m = jnp.max(x, axis=1, keepdims=True)
    e = jnp.exp(x - m)
    o_ref[...] = x - m - jnp.log(jnp.sum(e, axis=1, keepdims=True))


def _lsm(zk, deg2):
    return pl.pallas_call(
        _lsm_body,
        out_shape=jax.ShapeDtypeStruct((N_NODES, NCLS), jnp.float32),
    )(zk, deg2)


def kernel(x, edge_index, W1, b1, W2, b2):
    h = _mlp(x, W1, b1, W2, b2)

    src = jnp.asarray(edge_index[0], jnp.int32)
    dst = jnp.asarray(edge_index[1], jnp.int32)
    pad = jnp.full((E_PAD - E,), NPAD - 1, jnp.int32)
    src3 = jnp.concatenate([src, pad]).reshape(NTILES, CH_PT, CHUNK)
    dst3 = jnp.concatenate([dst, pad]).reshape(NTILES, CH_PT, CHUNK)

    zk, deg = _prop(h, src3, dst3)
    return _lsm(zk[:N_NODES], deg[:N_NODES].reshape(N_NODES, 1))


# speedup vs baseline: 1.9921x; 1.1305x over previous
import functools

import jax
import jax.numpy as jnp
from jax import lax
from jax.experimental import pallas as pl
from jax.experimental.pallas import tpu as pltpu
from jax.experimental.pallas import tpu_sc as plsc

N_NODES = 10000
D_FEAT = 128
HIDDEN = 64
NCLS = 16
K_HOPS = 10
ALPHA = 0.1
OMA = 1.0 - ALPHA

NPAD = 10240
NTILES = 16
ROWS_PT = NPAD // NTILES
HALF = ROWS_PT // 2
E = 320000
CHUNK = 256
CH_PT = 80
E_PT = CH_PT * CHUNK
E_PAD = NTILES * E_PT
V16 = 16
KB = 2
NDROWS = NPAD // V16
DROWS_PT = NDROWS // NTILES
DCHUNK = 128
DCH = NDROWS // DCHUNK


def _mlp_body(x_ref, w1_ref, b1_ref, w2_ref, b2_ref, h_ref):
    a = jnp.dot(x_ref[...], w1_ref[...], preferred_element_type=jnp.float32)
    a = jnp.maximum(a + b1_ref[...][None, :], 0.0)
    h_ref[...] = (
        jnp.dot(a, w2_ref[...], preferred_element_type=jnp.float32)
        + b2_ref[...][None, :]
    )


def _mlp(xp, W1, b1, W2, b2):
    return pl.pallas_call(
        _mlp_body,
        out_shape=jax.ShapeDtypeStruct((NPAD, NCLS), jnp.float32),
    )(xp, W1, b1, W2, b2)


_SC_MESH = plsc.VectorSubcoreMesh(core_axis_name="c", subcore_axis_name="s")


@functools.partial(
    pl.kernel,
    out_type=[
        jax.ShapeDtypeStruct((NPAD, NCLS), jnp.float32),
        jax.ShapeDtypeStruct((NPAD,), jnp.float32),
    ],
    mesh=_SC_MESH,
    compiler_params=pltpu.CompilerParams(
        needs_layout_passes=False, use_tc_tiling_on_sc=False),
    scratch_types=[
        pltpu.VMEM_SHARED((NPAD, NCLS), jnp.float32),
        pltpu.VMEM_SHARED((NPAD, NCLS), jnp.float32),
        pltpu.VMEM_SHARED((NDROWS, V16), jnp.float32),
        pltpu.VMEM((CH_PT, CHUNK), jnp.int32),
        pltpu.VMEM((CH_PT, CHUNK), jnp.int32),
        pltpu.VMEM((ROWS_PT, NCLS), jnp.float32),
        pltpu.VMEM((ROWS_PT, NCLS), jnp.float32),
        pltpu.VMEM((ROWS_PT, NCLS), jnp.float32),
        pltpu.VMEM((ROWS_PT, NCLS), jnp.float32),
        pltpu.VMEM((2 * KB, CHUNK, NCLS), jnp.float32),
        pltpu.VMEM((DCH, DCHUNK), jnp.int32),
        pltpu.VMEM((DROWS_PT, V16), jnp.float32),
        pltpu.VMEM((ROWS_PT,), jnp.float32),
        pltpu.VMEM((ROWS_PT,), jnp.float32),
        pltpu.SemaphoreType.DMA,
        pltpu.SemaphoreType.DMA((2,)),
    ],
)
def _prop(h_hbm, src_hbm, dst_hbm, zk_hbm, deg_hbm,
          z_sh, agg_sh, deg_sh,
          src_t, dst_t, hbuf, cbuf, gbuf, zbuf, ebuf, identx, dtmp, tmp,
          dinvb, gsem, ssem):
    abuf = hbuf
    cid = lax.axis_index("c")
    tid = lax.axis_index("s")
    row0 = tid * ROWS_PT

    ones16 = jnp.ones((V16,), jnp.float32)
    zeros16 = jnp.zeros((V16,), jnp.float32)
    iota16 = lax.iota(jnp.int32, V16)

    pltpu.sync_copy(src_hbm.at[tid], src_t)
    pltpu.sync_copy(dst_hbm.at[tid], dst_t)
    pltpu.sync_copy(h_hbm.at[pl.ds(row0, ROWS_PT)], hbuf)

    for c in range(DCH):
        for j in range(DCHUNK // V16):
            identx[c, pl.ds(j * V16, V16)] = iota16 + (c * DCHUNK + j * V16)

    def _zero_hist(r, _):
        cbuf[r, :] = zeros16
        return 0
    lax.fori_loop(0, ROWS_PT, _zero_hist, 0)

    def _zero_own(r, _):
        dtmp[r, :] = zeros16
        return 0
    lax.fori_loop(0, DROWS_PT, _zero_own, 0)
    pltpu.sync_copy(dtmp, deg_sh.at[pl.ds(tid * DROWS_PT, DROWS_PT)])

    def _count(i, _):
        for j in range(CHUNK // V16):
            idx = dst_t[i, pl.ds(j * V16, V16)]
            plsc.addupdate_scatter(cbuf, [idx >> 4, idx & 15], ones16)
        return 0
    lax.fori_loop(0, CH_PT, _count, 0)

    plsc.subcore_barrier()
    for c in range(DCH):
        pltpu.sync_copy(cbuf.at[pl.ds(c * DCHUNK, DCHUNK)],
                        deg_sh.at[identx.at[c]], add=True)
    plsc.subcore_barrier()

    pltpu.sync_copy(deg_sh.at[pl.ds(tid * DROWS_PT, DROWS_PT)], dtmp)

    def _deg_own(r, _):
        tmp[pl.ds(r * V16, V16)] = dtmp[r, :] + 1.0
        return 0
    lax.fori_loop(0, DROWS_PT, _deg_own, 0)

    off = cid * HALF
    pltpu.sync_copy(tmp.at[pl.ds(off, HALF)], deg_hbm.at[pl.ds(row0 + off, HALF)])

    def _rsqrt(i, _):
        s = pl.ds(i * V16, V16)
        d = tmp[s]
        yi = jnp.int32(0x5F3759DF) - (plsc.bitcast(d, jnp.int32) >> 1)
        y = plsc.bitcast(yi, jnp.float32)
        hx = d * 0.5
        y = y * (1.5 - hx * y * y)
        y = y * (1.5 - hx * y * y)
        y = y * (1.5 - hx * y * y)
        dinvb[s] = y
        return 0
    lax.fori_loop(0, ROWS_PT // V16, _rsqrt, 0)

    def _init_row(r, _):
        iv = jnp.full((V16,), r, jnp.int32)
        dv = plsc.load_gather(dinvb, [iv])
        hrow = hbuf[r, :]
        z0 = dv * hrow
        zbuf[r, :] = z0
        cbuf[r, :] = (dv * dv) * OMA
        gbuf[r, :] = z0 * ALPHA
        return 0
    lax.fori_loop(0, ROWS_PT, _init_row, 0)

    pltpu.sync_copy(zbuf, z_sh.at[pl.ds(row0, ROWS_PT)])
    pltpu.sync_copy(zbuf, agg_sh.at[pl.ds(row0, ROWS_PT)])

    for k in range(K_HOPS):
        plsc.subcore_barrier()

        def _pair(gg, _):
            for p in range(2):
                g = gg * 2 + p

                @pl.when(gg > 0)
                def _drain_prev():
                    for b in range(KB):
                        pltpu.make_async_copy(
                            h_hbm.at[pl.ds(0, CHUNK)],
                            ebuf.at[p * KB + b], ssem.at[p]).wait()

                handles = []
                for b in range(KB):
                    c = g * KB + b
                    handles.append(pltpu.async_copy(
                        z_sh.at[src_t.at[c]], ebuf.at[p * KB + b], gsem))
                for h in handles:
                    h.wait()
                for b in range(KB):
                    c = g * KB + b
                    pltpu.async_copy(ebuf.at[p * KB + b],
                                     agg_sh.at[dst_t.at[c]], ssem.at[p],
                                     add=True)
            return 0
        lax.fori_loop(0, CH_PT // (2 * KB), _pair, 0)

        for p in range(2):
            for b in range(KB):
                pltpu.make_async_copy(h_hbm.at[pl.ds(0, CHUNK)],
                                      ebuf.at[p * KB + b], ssem.at[p]).wait()

        plsc.subcore_barrier()

        pltpu.sync_copy(agg_sh.at[pl.ds(row0, ROWS_PT)], abuf)

        def _dense(r4, _):
            for u in range(4):
                r = r4 * 4 + u
                zbuf[r, :] = cbuf[r, :] * abuf[r, :] + gbuf[r, :]
            return 0
        lax.fori_loop(0, ROWS_PT // 4, _dense, 0)

        if k + 1 < K_HOPS:
            w1 = pltpu.async_copy(zbuf, z_sh.at[pl.ds(row0, ROWS_PT)], gsem)
            w2 = pltpu.async_copy(zbuf, agg_sh.at[pl.ds(row0, ROWS_PT)], gsem)
            w1.wait()
            w2.wait()

    pltpu.sync_copy(zbuf.at[pl.ds(off, HALF)], zk_hbm.at[pl.ds(row0 + off, HALF)])


def _lsm_body(z_ref, deg_ref, o_ref):
    x = z_ref[...] * jnp.sqrt(deg_ref[...])
    m = jnp.max(x, axis=1, keepdims=True)
    e = jnp.exp(x - m)
    o_ref[...] = x - m - jnp.log(jnp.sum(e, axis=1, keepdims=True))


def _lsm(zk, deg2):
    return pl.pallas_call(
        _lsm_body,
        out_shape=jax.ShapeDtypeStruct((N_NODES, NCLS), jnp.float32),
    )(zk, deg2)


def kernel(x, edge_index, W1, b1, W2, b2):
    xp = jnp.pad(x, ((0, NPAD - N_NODES), (0, 0)))
    h = _mlp(xp, W1, b1, W2, b2)

    src = jnp.asarray(edge_index[0], jnp.int32)
    dst = jnp.asarray(edge_index[1], jnp.int32)
    pad = jnp.full((E_PAD - E,), NPAD - 1, jnp.int32)
    src3 = jnp.concatenate([src, pad]).reshape(NTILES, CH_PT, CHUNK)
    dst3 = jnp.concatenate([dst, pad]).reshape(NTILES, CH_PT, CHUNK)

    zk, deg = _prop(h, src3, dst3)
    return _lsm(zk[:N_NODES], deg[:N_NODES].reshape(N_NODES, 1))
